# Initial kernel scaffold; baseline (speedup 1.0000x reference)
import functools

import jax
import jax.numpy as jnp
from jax import lax
from jax.experimental import pallas as pl
from jax.experimental.pallas import tpu as pltpu
from jax.experimental.pallas import tpu_sc as plsc

N = 10000
E = 320000
DIN = 128
DH = 256
HALF = 128
CORES = 2
TILES = 16
LANES = 16
K = 80
RPT = N // TILES
EPT_MP = E // TILES
EPT_DEG = E // (TILES * CORES)

_MESH = plsc.VectorSubcoreMesh(core_axis_name="c", subcore_axis_name="s")


def _deg_body(dst_hbm, ones_hbm, zeros_hbm, out_hbm, obuf, didx, deg_sp):
    c = lax.axis_index("c")
    s = lax.axis_index("s")
    wid = s * CORES + c
    pltpu.sync_copy(zeros_hbm, deg_sp.at[pl.ds(s * RPT, RPT)])
    pltpu.sync_copy(ones_hbm, obuf)
    plsc.subcore_barrier()
    base = wid * EPT_DEG

    def body(i, carry):
        off = base + i * K
        pltpu.sync_copy(dst_hbm.at[pl.ds(off, K)], didx)
        pltpu.sync_copy(obuf, deg_sp.at[didx], add=True)
        return carry

    lax.fori_loop(0, EPT_DEG // K, body, 0)
    plsc.subcore_barrier()
    pltpu.sync_copy(deg_sp.at[pl.ds(s * RPT, RPT)],
                    out_hbm.at[c, pl.ds(s * RPT, RPT)])


_deg_call = functools.partial(
    pl.kernel,
    out_type=jax.ShapeDtypeStruct((CORES, N, LANES), jnp.float32),
    mesh=_MESH,
    scratch_types=[
        pltpu.VMEM((K, LANES), jnp.float32),
        pltpu.VMEM((K,), jnp.int32),
        pltpu.VMEM_SHARED((N, LANES), jnp.float32),
    ],
)(_deg_body)


def _mp_body(ga_hbm, gb_hbm, src_hbm, dst_hbm, zeros_hbm,
             acca_hbm, accb_hbm, gbuf, sidx, didx, acc_sp, sem):
    c = lax.axis_index("c")
    s = lax.axis_index("s")
    pltpu.sync_copy(zeros_hbm, acc_sp.at[pl.ds(s * RPT, RPT)])
    plsc.subcore_barrier()
    base = s * EPT_MP

    def run(g_hbm, out_hbm):
        def body(i, carry):
            off = base + i * K
            pltpu.sync_copy(src_hbm.at[pl.ds(off, K)], sidx)
            pltpu.sync_copy(dst_hbm.at[pl.ds(off, K)], didx)
            pltpu.async_copy(g_hbm.at[sidx], gbuf, sem).wait()
            pltpu.sync_copy(gbuf, acc_sp.at[didx], add=True)
            return carry

        lax.fori_loop(0, EPT_MP // K, body, 0)
        plsc.subcore_barrier()
        pltpu.sync_copy(acc_sp.at[pl.ds(s * RPT, RPT)],
                        out_hbm.at[pl.ds(s * RPT, RPT)])

    @pl.when(c == 0)
    def _():
        run(ga_hbm, acca_hbm)

    @pl.when(c == 1)
    def _():
        run(gb_hbm, accb_hbm)


_mp_call = functools.partial(
    pl.kernel,
    out_type=[jax.ShapeDtypeStruct((N, HALF), jnp.float32),
              jax.ShapeDtypeStruct((N, HALF), jnp.float32)],
    mesh=_MESH,
    scratch_types=[
        pltpu.VMEM((K, HALF), jnp.float32),
        pltpu.VMEM((K,), jnp.int32),
        pltpu.VMEM((K,), jnp.int32),
        pltpu.VMEM_SHARED((N, HALF), jnp.float32),
        pltpu.SemaphoreType.DMA,
    ],
)(_mp_body)


EPT_DEC = E // (TILES * CORES)
DEC_ROWS = EPT_DEC // LANES


def _dec_body(u_hbm, v_hbm, src_hbm, dst_hbm, w2_hbm, bl2_hbm, out_hbm,
              ubuf, vbuf, sidx, didx, w2_v, bl2_v, score_v, sem_u, sem_v):
    c = lax.axis_index("c")
    s = lax.axis_index("s")
    wid = s * CORES + c
    base = wid * EPT_DEC
    pltpu.sync_copy(w2_hbm, w2_v)
    pltpu.sync_copy(bl2_hbm, bl2_v)
    bl2vec = bl2_v[...]

    def chunk(i, carry):
        off = base + i * K
        pltpu.sync_copy(src_hbm.at[pl.ds(off, K)], sidx)
        pltpu.sync_copy(dst_hbm.at[pl.ds(off, K)], didx)
        cu = pltpu.async_copy(u_hbm.at[sidx], ubuf, sem_u)
        cv = pltpu.async_copy(v_hbm.at[didx], vbuf, sem_v)
        cu.wait()
        cv.wait()

        def edge(e, carry2):
            acc = bl2vec
            for cc in range(DH // LANES):
                uc = ubuf[e, pl.ds(cc * LANES, LANES)]
                vc = vbuf[e, pl.ds(cc * LANES, LANES)]
                acc = acc + jnp.maximum(uc + vc, 0.0) * w2_v[cc]
            sc_val = jnp.sum(acc)
            idx = i * K + e
            score_v[idx // LANES, idx % LANES] = sc_val
            return carry2

        lax.fori_loop(0, K, edge, 0)
        return carry

    lax.fori_loop(0, EPT_DEC // K, chunk, 0)
    pltpu.sync_copy(score_v, out_hbm.at[wid])


_dec_call = functools.partial(
    pl.kernel,
    out_type=jax.ShapeDtypeStruct((TILES * CORES, DEC_ROWS, LANES),
                                  jnp.float32),
    mesh=_MESH,
    scratch_types=[
        pltpu.VMEM((K, DH), jnp.float32),
        pltpu.VMEM((K, DH), jnp.float32),
        pltpu.VMEM((K,), jnp.int32),
        pltpu.VMEM((K,), jnp.int32),
        pltpu.VMEM((DH // LANES, LANES), jnp.float32),
        pltpu.VMEM((LANES,), jnp.float32),
        pltpu.VMEM((DEC_ROWS, LANES), jnp.float32),
        pltpu.SemaphoreType.DMA,
        pltpu.SemaphoreType.DMA,
    ],
)(_dec_body)


R = 1000
GRID = N // R


def _mm1_body(x_ref, w_ref, degp_ref, ga_ref, gb_ref, dinv_ref):
    degp = degp_ref[...]
    deg = degp[0, :, 0:1] + degp[1, :, 0:1] + 1.0
    dinv = lax.rsqrt(deg)
    h = jnp.dot(x_ref[...], w_ref[...], preferred_element_type=jnp.float32)
    g = h * dinv
    ga_ref[...] = g[:, :HALF]
    gb_ref[...] = g[:, HALF:]
    dinv_ref[...] = dinv


def _mm1(x, w1, degp):
    return pl.pallas_call(
        _mm1_body,
        grid=(GRID,),
        in_specs=[
            pl.BlockSpec((R, DIN), lambda i: (i, 0)),
            pl.BlockSpec((DIN, DH), lambda i: (0, 0)),
            pl.BlockSpec((CORES, R, LANES), lambda i: (0, i, 0)),
        ],
        out_specs=[
            pl.BlockSpec((R, HALF), lambda i: (i, 0)),
            pl.BlockSpec((R, HALF), lambda i: (i, 0)),
            pl.BlockSpec((R, 1), lambda i: (i, 0)),
        ],
        out_shape=[
            jax.ShapeDtypeStruct((N, HALF), jnp.float32),
            jax.ShapeDtypeStruct((N, HALF), jnp.float32),
            jax.ShapeDtypeStruct((N, 1), jnp.float32),
        ],
    )(x, w1, degp)


def _mm23_body(acca_ref, accb_ref, ga_ref, gb_ref, dinv_ref, b_ref, w_ref,
               gao_ref, gbo_ref):
    dinv = dinv_ref[...]
    b = b_ref[...]
    za = jnp.maximum(dinv * (acca_ref[...] + ga_ref[...]) + b[:, :HALF], 0.0)
    zb = jnp.maximum(dinv * (accb_ref[...] + gb_ref[...]) + b[:, HALF:], 0.0)
    w = w_ref[...]
    h = (jnp.dot(za, w[:HALF, :], preferred_element_type=jnp.float32)
         + jnp.dot(zb, w[HALF:, :], preferred_element_type=jnp.float32))
    g = h * dinv
    gao_ref[...] = g[:, :HALF]
    gbo_ref[...] = g[:, HALF:]


def _mm23(acca, accb, ga, gb, dinv, b, w):
    return pl.pallas_call(
        _mm23_body,
        grid=(GRID,),
        in_specs=[
            pl.BlockSpec((R, HALF), lambda i: (i, 0)),
            pl.BlockSpec((R, HALF), lambda i: (i, 0)),
            pl.BlockSpec((R, HALF), lambda i: (i, 0)),
            pl.BlockSpec((R, HALF), lambda i: (i, 0)),
            pl.BlockSpec((R, 1), lambda i: (i, 0)),
            pl.BlockSpec((1, DH), lambda i: (0, 0)),
            pl.BlockSpec((DH, DH), lambda i: (0, 0)),
        ],
        out_specs=[
            pl.BlockSpec((R, HALF), lambda i: (i, 0)),
            pl.BlockSpec((R, HALF), lambda i: (i, 0)),
        ],
        out_shape=[
            jax.ShapeDtypeStruct((N, HALF), jnp.float32),
            jax.ShapeDtypeStruct((N, HALF), jnp.float32),
        ],
    )(acca, accb, ga, gb, dinv, b, w)


def _mm4_body(acca_ref, accb_ref, ga_ref, gb_ref, dinv_ref, b_ref, wl_ref,
              bl_ref, u_ref, v_ref):
    dinv = dinv_ref[...]
    b = b_ref[...]
    za = jnp.maximum(dinv * (acca_ref[...] + ga_ref[...]) + b[:, :HALF], 0.0)
    zb = jnp.maximum(dinv * (accb_ref[...] + gb_ref[...]) + b[:, HALF:], 0.0)
    wl = wl_ref[...]
    u = (jnp.dot(za, wl[:HALF, :], preferred_element_type=jnp.float32)
         + jnp.dot(zb, wl[HALF:DH, :], preferred_element_type=jnp.float32))
    v = (jnp.dot(za, wl[DH:DH + HALF, :], preferred_element_type=jnp.float32)
         + jnp.dot(zb, wl[DH + HALF:, :], preferred_element_type=jnp.float32))
    u_ref[...] = u + bl_ref[...]
    v_ref[...] = v


def _mm4(acca, accb, ga, gb, dinv, b, wl, bl):
    return pl.pallas_call(
        _mm4_body,
        grid=(GRID,),
        in_specs=[
            pl.BlockSpec((R, HALF), lambda i: (i, 0)),
            pl.BlockSpec((R, HALF), lambda i: (i, 0)),
            pl.BlockSpec((R, HALF), lambda i: (i, 0)),
            pl.BlockSpec((R, HALF), lambda i: (i, 0)),
            pl.BlockSpec((R, 1), lambda i: (i, 0)),
            pl.BlockSpec((1, DH), lambda i: (0, 0)),
            pl.BlockSpec((2 * DH, DH), lambda i: (0, 0)),
            pl.BlockSpec((1, DH), lambda i: (0, 0)),
        ],
        out_specs=[
            pl.BlockSpec((R, DH), lambda i: (i, 0)),
            pl.BlockSpec((R, DH), lambda i: (i, 0)),
        ],
        out_shape=[
            jax.ShapeDtypeStruct((N, DH), jnp.float32),
            jax.ShapeDtypeStruct((N, DH), jnp.float32),
        ],
    )(acca, accb, ga, gb, dinv, b, wl, bl)


def kernel(x, edge_index, W1, b1, W2, b2, W3, b3, Wl, bl, Wl2, bl2):
    src = edge_index[0]
    dst = edge_index[1]
    ones_deg = jnp.ones((K, LANES), jnp.float32)
    zeros_deg = jnp.zeros((RPT, LANES), jnp.float32)
    zeros_mp = jnp.zeros((RPT, HALF), jnp.float32)
    w2r = Wl2.reshape(DH // LANES, LANES)
    bl2v = jnp.full((LANES,), bl2[0] / LANES, jnp.float32)

    degp = _deg_call(dst, ones_deg, zeros_deg)
    ga1, gb1, dinv = _mm1(x, W1, degp)
    acca, accb = _mp_call(ga1, gb1, src, dst, zeros_mp)
    ga2, gb2 = _mm23(acca, accb, ga1, gb1, dinv, b1.reshape(1, DH), W2)
    acca, accb = _mp_call(ga2, gb2, src, dst, zeros_mp)
    ga3, gb3 = _mm23(acca, accb, ga2, gb2, dinv, b2.reshape(1, DH), W3)
    acca, accb = _mp_call(ga3, gb3, src, dst, zeros_mp)
    u, v = _mm4(acca, accb, ga3, gb3, dinv, b3.reshape(1, DH), Wl,
                bl.reshape(1, DH))
    raw = _dec_call(u, v, src, dst, w2r, bl2v)
    return raw.reshape(E, 1)


# speedup vs baseline: 6.1071x; 6.1071x over previous
import functools

import jax
import jax.numpy as jnp
from jax import lax
from jax.experimental import pallas as pl
from jax.experimental.pallas import tpu as pltpu
from jax.experimental.pallas import tpu_sc as plsc

N = 10000
E = 320000
DIN = 128
DH = 256
HALF = 128
CORES = 2
TILES = 16
LANES = 16
K = 80
NP = 10240
RPT = NP // TILES
EPT_MP = E // TILES
EPT_DEG = E // (TILES * CORES)

_MESH = plsc.VectorSubcoreMesh(core_axis_name="c", subcore_axis_name="s")


def _deg_body(dst_hbm, ones_hbm, zeros_hbm, out_hbm, obuf, didx, deg_sp):
    c = lax.axis_index("c")
    s = lax.axis_index("s")
    wid = s * CORES + c
    pltpu.sync_copy(zeros_hbm, deg_sp.at[pl.ds(s * RPT, RPT)])
    pltpu.sync_copy(ones_hbm, obuf)
    plsc.subcore_barrier()
    base = wid * EPT_DEG

    def body(i, carry):
        off = base + i * K
        pltpu.sync_copy(dst_hbm.at[pl.ds(off, K)], didx)
        pltpu.sync_copy(obuf, deg_sp.at[didx], add=True)
        return carry

    lax.fori_loop(0, EPT_DEG // K, body, 0)
    plsc.subcore_barrier()
    pltpu.sync_copy(deg_sp.at[pl.ds(s * RPT, RPT)],
                    out_hbm.at[c, pl.ds(s * RPT, RPT)])


_deg_call = functools.partial(
    pl.kernel,
    out_type=jax.ShapeDtypeStruct((CORES, NP, HALF), jnp.float32),
    mesh=_MESH,
    compiler_params=pltpu.CompilerParams(needs_layout_passes=False),
    scratch_types=[
        pltpu.VMEM((K, HALF), jnp.float32),
        pltpu.VMEM((K,), jnp.int32),
        pltpu.VMEM_SHARED((NP, HALF), jnp.float32),
    ],
)(_deg_body)


def _mp_body(ga_hbm, gb_hbm, src_hbm, dst_hbm, zeros_hbm,
             acca_hbm, accb_hbm, gbuf, sidx, didx, acc_sp, sem):
    c = lax.axis_index("c")
    s = lax.axis_index("s")
    pltpu.sync_copy(zeros_hbm, acc_sp.at[pl.ds(s * RPT, RPT)])
    plsc.subcore_barrier()
    base = s * EPT_MP

    def run(g_hbm, out_hbm):
        def body(i, carry):
            off = base + i * K
            pltpu.sync_copy(src_hbm.at[pl.ds(off, K)], sidx)
            pltpu.sync_copy(dst_hbm.at[pl.ds(off, K)], didx)
            pltpu.async_copy(g_hbm.at[sidx], gbuf, sem).wait()
            pltpu.sync_copy(gbuf, acc_sp.at[didx], add=True)
            return carry

        lax.fori_loop(0, EPT_MP // K, body, 0)
        plsc.subcore_barrier()
        pltpu.sync_copy(acc_sp.at[pl.ds(s * RPT, RPT)],
                        out_hbm.at[pl.ds(s * RPT, RPT)])

    @pl.when(c == 0)
    def _():
        run(ga_hbm, acca_hbm)

    @pl.when(c == 1)
    def _():
        run(gb_hbm, accb_hbm)


_mp_call = functools.partial(
    pl.kernel,
    out_type=[jax.ShapeDtypeStruct((NP, HALF), jnp.float32),
              jax.ShapeDtypeStruct((NP, HALF), jnp.float32)],
    mesh=_MESH,
    compiler_params=pltpu.CompilerParams(needs_layout_passes=False),
    scratch_types=[
        pltpu.VMEM((K, HALF), jnp.float32),
        pltpu.VMEM((K,), jnp.int32),
        pltpu.VMEM((K,), jnp.int32),
        pltpu.VMEM_SHARED((NP, HALF), jnp.float32),
        pltpu.SemaphoreType.DMA,
    ],
)(_mp_body)


EPT_DEC = E // (TILES * CORES)
DEC_ROWS = EPT_DEC // LANES


def _dec_body(u_hbm, v_hbm, src_hbm, dst_hbm, w2_hbm, bl2_hbm, out_hbm,
              ubuf, vbuf, sidx, didx, w2_v, bl2_v, score_v, tbuf,
              sem_u, sem_v):
    c = lax.axis_index("c")
    s = lax.axis_index("s")
    wid = s * CORES + c
    base = wid * EPT_DEC
    pltpu.sync_copy(w2_hbm, w2_v)
    pltpu.sync_copy(bl2_hbm, bl2_v)
    bl2vec = bl2_v[...]

    def chunk(i, carry):
        off = base + i * K
        pltpu.sync_copy(src_hbm.at[pl.ds(off, K)], sidx)
        pltpu.sync_copy(dst_hbm.at[pl.ds(off, K)], didx)
        cu = pltpu.async_copy(u_hbm.at[sidx], ubuf, sem_u)
        cv = pltpu.async_copy(v_hbm.at[didx], vbuf, sem_v)
        cu.wait()
        cv.wait()

        iota16 = lax.iota(jnp.int32, LANES)

        def group(gidx, carry2):
            e_base = gidx * LANES
            for j in range(LANES):
                e = e_base + j
                acc = bl2vec
                for cc in range(DH // LANES):
                    uc = ubuf[e, pl.ds(cc * LANES, LANES)]
                    vc = vbuf[e, pl.ds(cc * LANES, LANES)]
                    acc = acc + jnp.maximum(uc + vc, 0.0) * w2_v[cc]
                tbuf[j] = acc
            svec = jnp.zeros((LANES,), jnp.float32)
            for j in range(LANES):
                svec = svec + plsc.load_gather(
                    tbuf, [iota16, jnp.full((LANES,), j, jnp.int32)])
            score_v[i * (K // LANES) + gidx] = svec
            return carry2

        lax.fori_loop(0, K // LANES, group, 0)
        return carry

    lax.fori_loop(0, EPT_DEC // K, chunk, 0)
    pltpu.sync_copy(score_v, out_hbm.at[wid])


_dec_call = functools.partial(
    pl.kernel,
    out_type=jax.ShapeDtypeStruct((TILES * CORES, DEC_ROWS, LANES),
                                  jnp.float32),
    mesh=_MESH,
    compiler_params=pltpu.CompilerParams(needs_layout_passes=False),
    scratch_types=[
        pltpu.VMEM((K, DH), jnp.float32),
        pltpu.VMEM((K, DH), jnp.float32),
        pltpu.VMEM((K,), jnp.int32),
        pltpu.VMEM((K,), jnp.int32),
        pltpu.VMEM((DH // LANES, LANES), jnp.float32),
        pltpu.VMEM((LANES,), jnp.float32),
        pltpu.VMEM((DEC_ROWS, LANES), jnp.float32),
        pltpu.VMEM((LANES, LANES), jnp.float32),
        pltpu.SemaphoreType.DMA,
        pltpu.SemaphoreType.DMA,
    ],
)(_dec_body)


R = 1000
GRID = N // R


def _mm1_body(x_ref, w_ref, degp_ref, ga_ref, gb_ref, dinv_ref):
    degp = degp_ref[...]
    deg = degp[0, :, 0:1] + degp[1, :, 0:1] + 1.0
    dinv = lax.rsqrt(deg)
    h = jnp.dot(x_ref[...], w_ref[...], preferred_element_type=jnp.float32,
             precision=lax.Precision.HIGHEST)
    g = h * dinv
    ga_ref[...] = g[:, :HALF]
    gb_ref[...] = g[:, HALF:]
    dinv_ref[...] = dinv


def _mm1(x, w1, degp):
    return pl.pallas_call(
        _mm1_body,
        grid=(GRID,),
        in_specs=[
            pl.BlockSpec((R, DIN), lambda i: (i, 0)),
            pl.BlockSpec((DIN, DH), lambda i: (0, 0)),
            pl.BlockSpec((CORES, R, HALF), lambda i: (0, i, 0)),
        ],
        out_specs=[
            pl.BlockSpec((R, HALF), lambda i: (i, 0)),
            pl.BlockSpec((R, HALF), lambda i: (i, 0)),
            pl.BlockSpec((R, 1), lambda i: (i, 0)),
        ],
        out_shape=[
            jax.ShapeDtypeStruct((N, HALF), jnp.float32),
            jax.ShapeDtypeStruct((N, HALF), jnp.float32),
            jax.ShapeDtypeStruct((N, 1), jnp.float32),
        ],
    )(x, w1, degp)


def _mm23_body(acca_ref, accb_ref, ga_ref, gb_ref, dinv_ref, b_ref, w_ref,
               gao_ref, gbo_ref):
    dinv = dinv_ref[...]
    b = b_ref[...]
    za = jnp.maximum(dinv * (acca_ref[...] + ga_ref[...]) + b[:, :HALF], 0.0)
    zb = jnp.maximum(dinv * (accb_ref[...] + gb_ref[...]) + b[:, HALF:], 0.0)
    w = w_ref[...]
    h = (jnp.dot(za, w[:HALF, :], preferred_element_type=jnp.float32,
             precision=lax.Precision.HIGHEST)
         + jnp.dot(zb, w[HALF:, :], preferred_element_type=jnp.float32,
             precision=lax.Precision.HIGHEST))
    g = h * dinv
    gao_ref[...] = g[:, :HALF]
    gbo_ref[...] = g[:, HALF:]


def _mm23(acca, accb, ga, gb, dinv, b, w):
    return pl.pallas_call(
        _mm23_body,
        grid=(GRID,),
        in_specs=[
            pl.BlockSpec((R, HALF), lambda i: (i, 0)),
            pl.BlockSpec((R, HALF), lambda i: (i, 0)),
            pl.BlockSpec((R, HALF), lambda i: (i, 0)),
            pl.BlockSpec((R, HALF), lambda i: (i, 0)),
            pl.BlockSpec((R, 1), lambda i: (i, 0)),
            pl.BlockSpec((1, DH), lambda i: (0, 0)),
            pl.BlockSpec((DH, DH), lambda i: (0, 0)),
        ],
        out_specs=[
            pl.BlockSpec((R, HALF), lambda i: (i, 0)),
            pl.BlockSpec((R, HALF), lambda i: (i, 0)),
        ],
        out_shape=[
            jax.ShapeDtypeStruct((N, HALF), jnp.float32),
            jax.ShapeDtypeStruct((N, HALF), jnp.float32),
        ],
    )(acca, accb, ga, gb, dinv, b, w)


def _mm4_body(acca_ref, accb_ref, ga_ref, gb_ref, dinv_ref, b_ref, wl_ref,
              bl_ref, u_ref, v_ref):
    dinv = dinv_ref[...]
    b = b_ref[...]
    za = jnp.maximum(dinv * (acca_ref[...] + ga_ref[...]) + b[:, :HALF], 0.0)
    zb = jnp.maximum(dinv * (accb_ref[...] + gb_ref[...]) + b[:, HALF:], 0.0)
    wl = wl_ref[...]
    u = (jnp.dot(za, wl[:HALF, :], preferred_element_type=jnp.float32,
             precision=lax.Precision.HIGHEST)
         + jnp.dot(zb, wl[HALF:DH, :], preferred_element_type=jnp.float32,
             precision=lax.Precision.HIGHEST))
    v = (jnp.dot(za, wl[DH:DH + HALF, :], preferred_element_type=jnp.float32,
             precision=lax.Precision.HIGHEST)
         + jnp.dot(zb, wl[DH + HALF:, :], preferred_element_type=jnp.float32,
             precision=lax.Precision.HIGHEST))
    u_ref[...] = u + bl_ref[...]
    v_ref[...] = v


def _mm4(acca, accb, ga, gb, dinv, b, wl, bl):
    return pl.pallas_call(
        _mm4_body,
        grid=(GRID,),
        in_specs=[
            pl.BlockSpec((R, HALF), lambda i: (i, 0)),
            pl.BlockSpec((R, HALF), lambda i: (i, 0)),
            pl.BlockSpec((R, HALF), lambda i: (i, 0)),
            pl.BlockSpec((R, HALF), lambda i: (i, 0)),
            pl.BlockSpec((R, 1), lambda i: (i, 0)),
            pl.BlockSpec((1, DH), lambda i: (0, 0)),
            pl.BlockSpec((2 * DH, DH), lambda i: (0, 0)),
            pl.BlockSpec((1, DH), lambda i: (0, 0)),
        ],
        out_specs=[
            pl.BlockSpec((R, DH), lambda i: (i, 0)),
            pl.BlockSpec((R, DH), lambda i: (i, 0)),
        ],
        out_shape=[
            jax.ShapeDtypeStruct((N, DH), jnp.float32),
            jax.ShapeDtypeStruct((N, DH), jnp.float32),
        ],
    )(acca, accb, ga, gb, dinv, b, wl, bl)


def kernel(x, edge_index, W1, b1, W2, b2, W3, b3, Wl, bl, Wl2, bl2):
    src = edge_index[0]
    dst = edge_index[1]
    ones_deg = jnp.ones((K, HALF), jnp.float32)
    zeros_deg = jnp.zeros((RPT, HALF), jnp.float32)
    zeros_mp = jnp.zeros((RPT, HALF), jnp.float32)
    w2r = Wl2.reshape(DH // LANES, LANES)
    bl2v = jnp.full((LANES,), bl2[0] / LANES, jnp.float32)

    degp = _deg_call(dst, ones_deg, zeros_deg)
    ga1, gb1, dinv = _mm1(x, W1, degp)
    acca, accb = _mp_call(ga1, gb1, src, dst, zeros_mp)
    ga2, gb2 = _mm23(acca, accb, ga1, gb1, dinv, b1.reshape(1, DH), W2)
    acca, accb = _mp_call(ga2, gb2, src, dst, zeros_mp)
    ga3, gb3 = _mm23(acca, accb, ga2, gb2, dinv, b2.reshape(1, DH), W3)
    acca, accb = _mp_call(ga3, gb3, src, dst, zeros_mp)
    u, v = _mm4(acca, accb, ga3, gb3, dinv, b3.reshape(1, DH), Wl,
                bl.reshape(1, DH))
    raw = _dec_call(u, v, src, dst, w2r, bl2v)
    return raw.reshape(E, 1)


# speedup vs baseline: 10.0423x; 1.6444x over previous
import functools

import jax
import jax.numpy as jnp
from jax import lax
from jax.experimental import pallas as pl
from jax.experimental.pallas import tpu as pltpu
from jax.experimental.pallas import tpu_sc as plsc

N = 10000
E = 320000
DIN = 128
DH = 256
HALF = 128
CORES = 2
TILES = 16
LANES = 16
K = 80
NP = 10240
RPT = NP // TILES
CPT_MP = E // (TILES * K)
CPT_32 = E // (TILES * CORES * K)

_MESH = plsc.VectorSubcoreMesh(core_axis_name="c", subcore_axis_name="s")
_PARAMS = pltpu.CompilerParams(needs_layout_passes=False)


def _deg_body(dst32_hbm, ones_hbm, zeros_hbm, out_hbm, obuf, didx_t, deg_sp,
              sem):
    c = lax.axis_index("c")
    s = lax.axis_index("s")
    wid = s * CORES + c
    pltpu.sync_copy(zeros_hbm, deg_sp.at[pl.ds(s * RPT, RPT)])
    pltpu.sync_copy(ones_hbm, obuf)
    plsc.subcore_barrier()

    BATCH = 5

    def stage(hh, carry0):
        pltpu.sync_copy(dst32_hbm.at[wid, hh], didx_t)

        def body(b, carry):
            i0 = b * BATCH
            for t in range(BATCH):
                pltpu.async_copy(obuf, deg_sp.at[didx_t.at[i0 + t]], sem,
                                 add=True)
            for t in range(BATCH):
                pltpu.make_async_copy(obuf, deg_sp.at[didx_t.at[i0 + t]],
                                      sem).wait()
            return carry

        lax.fori_loop(0, 25 // BATCH, body, 0)
        return carry0

    lax.fori_loop(0, 5, stage, 0)
    plsc.subcore_barrier()
    pltpu.sync_copy(deg_sp.at[pl.ds(s * RPT, RPT)],
                    out_hbm.at[c, pl.ds(s * RPT, RPT)])


_deg_call = functools.partial(
    pl.kernel,
    out_type=jax.ShapeDtypeStruct((CORES, NP, HALF), jnp.float32),
    mesh=_MESH,
    compiler_params=_PARAMS,
    scratch_types=[
        pltpu.VMEM((K, HALF), jnp.float32),
        pltpu.VMEM((25, K), jnp.int32),
        pltpu.VMEM_SHARED((NP, HALF), jnp.float32),
        pltpu.SemaphoreType.DMA,
    ],
)(_deg_body)


NSTAGE = 5
HCPT = CPT_MP // NSTAGE


def _mp_body(ga_hbm, gb_hbm, src16_hbm, dst16_hbm, zeros_hbm,
             acca_hbm, accb_hbm, sidx_t, didx_t, buf_a, buf_b, acc_sp,
             gs_a, gs_b):
    c = lax.axis_index("c")
    s = lax.axis_index("s")
    pltpu.sync_copy(zeros_hbm, acc_sp.at[pl.ds(s * RPT, RPT)])
    plsc.subcore_barrier()

    def run(g_hbm, out_hbm):
        def half(h, carry0):
            pltpu.sync_copy(src16_hbm.at[s, h], sidx_t)
            pltpu.sync_copy(dst16_hbm.at[s, h], didx_t)
            pltpu.async_copy(g_hbm.at[sidx_t.at[0]], buf_a, gs_a)

            def chunk(i, carry):
                even = jnp.bitwise_and(i, 1) == 0

                @pl.when(even)
                def _():
                    pltpu.make_async_copy(g_hbm.at[sidx_t.at[i]], buf_a,
                                          gs_a).wait()

                    @pl.when(i + 1 < HCPT)
                    def _():
                        pltpu.async_copy(g_hbm.at[sidx_t.at[i + 1]], buf_b,
                                         gs_b)

                    pltpu.sync_copy(buf_a, acc_sp.at[didx_t.at[i]], add=True)

                @pl.when(jnp.logical_not(even))
                def _():
                    pltpu.make_async_copy(g_hbm.at[sidx_t.at[i]], buf_b,
                                          gs_b).wait()

                    @pl.when(i + 1 < HCPT)
                    def _():
                        pltpu.async_copy(g_hbm.at[sidx_t.at[i + 1]], buf_a,
                                         gs_a)

                    pltpu.sync_copy(buf_b, acc_sp.at[didx_t.at[i]], add=True)

                return carry

            lax.fori_loop(0, HCPT, chunk, 0)
            return carry0

        lax.fori_loop(0, NSTAGE, half, 0)
        plsc.subcore_barrier()
        pltpu.sync_copy(acc_sp.at[pl.ds(s * RPT, RPT)],
                        out_hbm.at[pl.ds(s * RPT, RPT)])

    @pl.when(c == 0)
    def _():
        run(ga_hbm, acca_hbm)

    @pl.when(c == 1)
    def _():
        run(gb_hbm, accb_hbm)


_mp_call = functools.partial(
    pl.kernel,
    out_type=[jax.ShapeDtypeStruct((NP, HALF), jnp.float32),
              jax.ShapeDtypeStruct((NP, HALF), jnp.float32)],
    mesh=_MESH,
    compiler_params=_PARAMS,
    scratch_types=[
        pltpu.VMEM((HCPT, K), jnp.int32),
        pltpu.VMEM((HCPT, K), jnp.int32),
        pltpu.VMEM((K, HALF), jnp.float32),
        pltpu.VMEM((K, HALF), jnp.float32),
        pltpu.VMEM_SHARED((NP, HALF), jnp.float32),
        pltpu.SemaphoreType.DMA,
        pltpu.SemaphoreType.DMA,
    ],
)(_mp_body)


EPT_DEC = E // (TILES * CORES)
DEC_ROWS = EPT_DEC // LANES


DEC_STAGES = 5
KD = LANES
DEC_CPT = EPT_DEC // KD
DEC_SCPT = DEC_CPT // DEC_STAGES


def _dec_body(u_hbm, v_hbm, src32_hbm, dst32_hbm, w2_hbm, bl2_hbm, out_hbm,
              ub_a, ub_b, vb_a, vb_b, sidx_t, didx_t, w2_v, bl2_v, score_v,
              tbuf, sem_ua, sem_ub, sem_va, sem_vb):
    c = lax.axis_index("c")
    s = lax.axis_index("s")
    wid = s * CORES + c
    pltpu.sync_copy(w2_hbm, w2_v)
    pltpu.sync_copy(bl2_hbm, bl2_v)
    bl2vec = bl2_v[...]
    iota16 = lax.iota(jnp.int32, LANES)

    def compute(i, ubr, vbr):
        def edge(j, carry2):
            zeros16 = jnp.zeros((LANES,), jnp.float32)
            accs = [bl2vec, zeros16, zeros16, zeros16]
            for cc in range(DH // LANES):
                uc = ubr[j, pl.ds(cc * LANES, LANES)]
                vc = vbr[j, pl.ds(cc * LANES, LANES)]
                k = cc % 4
                accs[k] = accs[k] + jnp.maximum(uc + vc, 0.0) * w2_v[cc]
            tbuf[j] = (accs[0] + accs[1]) + (accs[2] + accs[3])
            return carry2

        lax.fori_loop(0, LANES, edge, 0)
        svec = jnp.zeros((LANES,), jnp.float32)
        for j in range(LANES):
            svec = svec + plsc.load_gather(
                tbuf, [iota16, jnp.full((LANES,), j, jnp.int32)])
        score_v[i] = svec

    def stage(hh, carry0):
        pltpu.sync_copy(src32_hbm.at[wid, hh], sidx_t)
        pltpu.sync_copy(dst32_hbm.at[wid, hh], didx_t)
        pltpu.async_copy(u_hbm.at[sidx_t.at[0]], ub_a, sem_ua)
        pltpu.async_copy(v_hbm.at[didx_t.at[0]], vb_a, sem_va)

        def chunk(i, carry):
            even = jnp.bitwise_and(i, 1) == 0

            @pl.when(even)
            def _():
                pltpu.make_async_copy(u_hbm.at[sidx_t.at[i]], ub_a,
                                      sem_ua).wait()
                pltpu.make_async_copy(v_hbm.at[didx_t.at[i]], vb_a,
                                      sem_va).wait()

                @pl.when(i + 1 < DEC_SCPT)
                def _():
                    pltpu.async_copy(u_hbm.at[sidx_t.at[i + 1]], ub_b, sem_ub)
                    pltpu.async_copy(v_hbm.at[didx_t.at[i + 1]], vb_b, sem_vb)

                compute(i, ub_a, vb_a)

            @pl.when(jnp.logical_not(even))
            def _():
                pltpu.make_async_copy(u_hbm.at[sidx_t.at[i]], ub_b,
                                      sem_ub).wait()
                pltpu.make_async_copy(v_hbm.at[didx_t.at[i]], vb_b,
                                      sem_vb).wait()

                @pl.when(i + 1 < DEC_SCPT)
                def _():
                    pltpu.async_copy(u_hbm.at[sidx_t.at[i + 1]], ub_a, sem_ua)
                    pltpu.async_copy(v_hbm.at[didx_t.at[i + 1]], vb_a, sem_va)

                compute(i, ub_b, vb_b)

            return carry

        lax.fori_loop(0, DEC_SCPT, chunk, 0)
        pltpu.sync_copy(score_v, out_hbm.at[wid, hh])
        return carry0

    lax.fori_loop(0, DEC_STAGES, stage, 0)


_dec_call = functools.partial(
    pl.kernel,
    out_type=jax.ShapeDtypeStruct(
        (TILES * CORES, DEC_STAGES, DEC_SCPT, LANES), jnp.float32),
    mesh=_MESH,
    compiler_params=_PARAMS,
    scratch_types=[
        pltpu.VMEM((KD, DH), jnp.float32),
        pltpu.VMEM((KD, DH), jnp.float32),
        pltpu.VMEM((KD, DH), jnp.float32),
        pltpu.VMEM((KD, DH), jnp.float32),
        pltpu.VMEM((DEC_SCPT, KD), jnp.int32),
        pltpu.VMEM((DEC_SCPT, KD), jnp.int32),
        pltpu.VMEM((DH // LANES, LANES), jnp.float32),
        pltpu.VMEM((LANES,), jnp.float32),
        pltpu.VMEM((DEC_SCPT, LANES), jnp.float32),
        pltpu.VMEM((LANES, LANES), jnp.float32),
        pltpu.SemaphoreType.DMA,
        pltpu.SemaphoreType.DMA,
        pltpu.SemaphoreType.DMA,
        pltpu.SemaphoreType.DMA,
    ],
)(_dec_body)


R = 1000
GRID = N // R


def _mm1_body(x_ref, w_ref, degp_ref, ga_ref, gb_ref, dinv_ref):
    degp = degp_ref[...]
    deg = degp[0, :, 0:1] + degp[1, :, 0:1] + 1.0
    dinv = lax.rsqrt(deg)
    h = jnp.dot(x_ref[...], w_ref[...], preferred_element_type=jnp.float32,
                precision=lax.Precision.HIGHEST)
    g = h * dinv
    ga_ref[...] = g[:, :HALF]
    gb_ref[...] = g[:, HALF:]
    dinv_ref[...] = dinv


def _mm1(x, w1, degp):
    return pl.pallas_call(
        _mm1_body,
        grid=(GRID,),
        in_specs=[
            pl.BlockSpec((R, DIN), lambda i: (i, 0)),
            pl.BlockSpec((DIN, DH), lambda i: (0, 0)),
            pl.BlockSpec((CORES, R, HALF), lambda i: (0, i, 0)),
        ],
        out_specs=[
            pl.BlockSpec((R, HALF), lambda i: (i, 0)),
            pl.BlockSpec((R, HALF), lambda i: (i, 0)),
            pl.BlockSpec((R, 1), lambda i: (i, 0)),
        ],
        out_shape=[
            jax.ShapeDtypeStruct((N, HALF), jnp.float32),
            jax.ShapeDtypeStruct((N, HALF), jnp.float32),
            jax.ShapeDtypeStruct((N, 1), jnp.float32),
        ],
    )(x, w1, degp)


def _mm23_body(acca_ref, accb_ref, ga_ref, gb_ref, dinv_ref, b_ref, w_ref,
               gao_ref, gbo_ref):
    dinv = dinv_ref[...]
    b = b_ref[...]
    za = jnp.maximum(dinv * (acca_ref[...] + ga_ref[...]) + b[:, :HALF], 0.0)
    zb = jnp.maximum(dinv * (accb_ref[...] + gb_ref[...]) + b[:, HALF:], 0.0)
    w = w_ref[...]
    h = (jnp.dot(za, w[:HALF, :], preferred_element_type=jnp.float32,
                 precision=lax.Precision.HIGHEST)
         + jnp.dot(zb, w[HALF:, :], preferred_element_type=jnp.float32,
                   precision=lax.Precision.HIGHEST))
    g = h * dinv
    gao_ref[...] = g[:, :HALF]
    gbo_ref[...] = g[:, HALF:]


def _mm23(acca, accb, ga, gb, dinv, b, w):
    return pl.pallas_call(
        _mm23_body,
        grid=(GRID,),
        in_specs=[
            pl.BlockSpec((R, HALF), lambda i: (i, 0)),
            pl.BlockSpec((R, HALF), lambda i: (i, 0)),
            pl.BlockSpec((R, HALF), lambda i: (i, 0)),
            pl.BlockSpec((R, HALF), lambda i: (i, 0)),
            pl.BlockSpec((R, 1), lambda i: (i, 0)),
            pl.BlockSpec((1, DH), lambda i: (0, 0)),
            pl.BlockSpec((DH, DH), lambda i: (0, 0)),
        ],
        out_specs=[
            pl.BlockSpec((R, HALF), lambda i: (i, 0)),
            pl.BlockSpec((R, HALF), lambda i: (i, 0)),
        ],
        out_shape=[
            jax.ShapeDtypeStruct((N, HALF), jnp.float32),
            jax.ShapeDtypeStruct((N, HALF), jnp.float32),
        ],
    )(acca, accb, ga, gb, dinv, b, w)


def _mm4_body(acca_ref, accb_ref, ga_ref, gb_ref, dinv_ref, b_ref, wl_ref,
              bl_ref, u_ref, v_ref):
    dinv = dinv_ref[...]
    b = b_ref[...]
    za = jnp.maximum(dinv * (acca_ref[...] + ga_ref[...]) + b[:, :HALF], 0.0)
    zb = jnp.maximum(dinv * (accb_ref[...] + gb_ref[...]) + b[:, HALF:], 0.0)
    wl = wl_ref[...]
    u = (jnp.dot(za, wl[:HALF, :], preferred_element_type=jnp.float32,
                 precision=lax.Precision.HIGHEST)
         + jnp.dot(zb, wl[HALF:DH, :], preferred_element_type=jnp.float32,
                   precision=lax.Precision.HIGHEST))
    v = (jnp.dot(za, wl[DH:DH + HALF, :], preferred_element_type=jnp.float32,
                 precision=lax.Precision.HIGHEST)
         + jnp.dot(zb, wl[DH + HALF:, :], preferred_element_type=jnp.float32,
                   precision=lax.Precision.HIGHEST))
    u_ref[...] = u + bl_ref[...]
    v_ref[...] = v


def _mm4(acca, accb, ga, gb, dinv, b, wl, bl):
    return pl.pallas_call(
        _mm4_body,
        grid=(GRID,),
        in_specs=[
            pl.BlockSpec((R, HALF), lambda i: (i, 0)),
            pl.BlockSpec((R, HALF), lambda i: (i, 0)),
            pl.BlockSpec((R, HALF), lambda i: (i, 0)),
            pl.BlockSpec((R, HALF), lambda i: (i, 0)),
            pl.BlockSpec((R, 1), lambda i: (i, 0)),
            pl.BlockSpec((1, DH), lambda i: (0, 0)),
            pl.BlockSpec((2 * DH, DH), lambda i: (0, 0)),
            pl.BlockSpec((1, DH), lambda i: (0, 0)),
        ],
        out_specs=[
            pl.BlockSpec((R, DH), lambda i: (i, 0)),
            pl.BlockSpec((R, DH), lambda i: (i, 0)),
        ],
        out_shape=[
            jax.ShapeDtypeStruct((N, DH), jnp.float32),
            jax.ShapeDtypeStruct((N, DH), jnp.float32),
        ],
    )(acca, accb, ga, gb, dinv, b, wl, bl)


def kernel(x, edge_index, W1, b1, W2, b2, W3, b3, Wl, bl, Wl2, bl2):
    src = edge_index[0]
    dst = edge_index[1]
    src16 = src.reshape(TILES, NSTAGE, HCPT, K)
    dst16 = dst.reshape(TILES, NSTAGE, HCPT, K)
    src32 = src.reshape(TILES * CORES, DEC_STAGES, DEC_SCPT, KD)
    dst32 = dst.reshape(TILES * CORES, DEC_STAGES, DEC_SCPT, KD)
    ones_deg = jnp.ones((K, HALF), jnp.float32)
    zeros_mp = jnp.zeros((RPT, HALF), jnp.float32)
    w2r = Wl2.reshape(DH // LANES, LANES)
    bl2v = jnp.full((LANES,), bl2[0] / LANES, jnp.float32)

    dstdeg = dst.reshape(TILES * CORES, 5, 25, K)
    degp = _deg_call(dstdeg, ones_deg, zeros_mp)
    ga1, gb1, dinv = _mm1(x, W1, degp)
    acca, accb = _mp_call(ga1, gb1, src16, dst16, zeros_mp)
    ga2, gb2 = _mm23(acca, accb, ga1, gb1, dinv, b1.reshape(1, DH), W2)
    acca, accb = _mp_call(ga2, gb2, src16, dst16, zeros_mp)
    ga3, gb3 = _mm23(acca, accb, ga2, gb2, dinv, b2.reshape(1, DH), W3)
    acca, accb = _mp_call(ga3, gb3, src16, dst16, zeros_mp)
    u, v = _mm4(acca, accb, ga3, gb3, dinv, b3.reshape(1, DH), Wl,
                bl.reshape(1, DH))
    raw = _dec_call(u, v, src32, dst32, w2r, bl2v)
    return raw.reshape(E, 1)
